# Initial kernel scaffold; baseline (speedup 1.0000x reference)
import jax
import jax.numpy as jnp
from jax.experimental import pallas as pl


def kernel(vision_embed, text_embed):
    raise NotImplementedError("write your pallas kernel here")


# speedup vs baseline: 9.0805x; 9.0805x over previous
import functools

import jax
import jax.numpy as jnp
from jax.experimental import pallas as pl
from jax.experimental.pallas import tpu as pltpu

_TEMPERATURE = 0.07
_NUM_HARD = 4
_LANES = 128


def _loss_kernel(v_ref, t_ref, out_ref, colsum_ref, rowpart_ref, *, n_rows):
    i = pl.program_id(0)
    n_steps = pl.num_programs(0)

    v = v_ref[...]
    t = t_ref[...]
    s = jax.lax.dot_general(
        v, t, (((1,), (0,)), ((), ())),
        preferred_element_type=jnp.float32) * (1.0 / _TEMPERATURE)
    r, b = s.shape

    row_ids = i * r + jax.lax.broadcasted_iota(jnp.int32, (r, b), 0)
    col_ids = jax.lax.broadcasted_iota(jnp.int32, (r, b), 1)
    diag = row_ids == col_ids

    e = jnp.exp(s)
    s_ii = jnp.sum(jnp.where(diag, s, 0.0), axis=1)
    sm = jnp.where(diag, -1e9, s)
    for _ in range(_NUM_HARD):
        m = jnp.max(sm, axis=1, keepdims=True)
        idx = jnp.min(jnp.where(sm == m, col_ids, b), axis=1, keepdims=True)
        onehot = col_ids == idx
        e = jnp.where(onehot, e * e, e)
        sm = jnp.where(onehot, -1e9, sm)

    rowsum = jnp.sum(e, axis=1)
    part = jnp.sum(jnp.log(rowsum) - 2.0 * s_ii)
    part_vec = jnp.full((1, _LANES), part / _LANES, dtype=jnp.float32)
    colpart = jnp.sum(e, axis=0, keepdims=True)

    @pl.when(i == 0)
    def _init():
        colsum_ref[...] = colpart
        rowpart_ref[...] = part_vec

    @pl.when(i > 0)
    def _acc():
        colsum_ref[...] += colpart
        rowpart_ref[...] += part_vec

    @pl.when(i == n_steps - 1)
    def _final():
        total = (jnp.sum(jnp.log(colsum_ref[...]))
                 + jnp.sum(rowpart_ref[...]))
        out_ref[...] = jnp.full((1, _LANES), total / (2.0 * n_rows),
                                dtype=jnp.float32)


@jax.jit
def kernel(vision_embed, text_embed):
    b, d = vision_embed.shape
    block_r = 256
    grid = (b // block_r,)
    t_t = text_embed.T

    out = pl.pallas_call(
        functools.partial(_loss_kernel, n_rows=b),
        grid=grid,
        in_specs=[
            pl.BlockSpec((block_r, d), lambda i: (i, 0)),
            pl.BlockSpec((d, b), lambda i: (0, 0)),
        ],
        out_specs=pl.BlockSpec((1, _LANES), lambda i: (0, 0)),
        out_shape=jax.ShapeDtypeStruct((1, _LANES), jnp.float32),
        scratch_shapes=[
            pltpu.VMEM((1, b), jnp.float32),
            pltpu.VMEM((1, _LANES), jnp.float32),
        ],
    )(vision_embed, t_t)
    return out[0, 0]


# speedup vs baseline: 18.6412x; 2.0529x over previous
import functools

import jax
import jax.numpy as jnp
from jax.experimental import pallas as pl
from jax.experimental.pallas import tpu as pltpu

_TEMPERATURE = 0.07
_NUM_HARD = 4
_LANES = 128


def _loss_kernel(v_ref, t_ref, out_ref, colsum_ref, rowpart_ref, s_scr,
                 *, n_rows):
    i = pl.program_id(0)
    n_steps = pl.num_programs(0)

    v = v_ref[...]
    t = t_ref[...]
    s = jax.lax.dot_general(
        v, t, (((1,), (0,)), ((), ())),
        preferred_element_type=jnp.float32)
    r, b = s.shape

    col0 = i * r
    s_scr[...] = s
    s_win = s_scr[:, pl.ds(col0, r)]
    dmask = (jax.lax.broadcasted_iota(jnp.int32, (r, r), 0)
             == jax.lax.broadcasted_iota(jnp.int32, (r, r), 1))
    s_ii = jnp.sum(jnp.where(dmask, s_win, 0.0), axis=1)
    s_scr[:, pl.ds(col0, r)] = jnp.where(dmask, -1e9, s_win)
    sm0 = s_scr[...]

    m = jnp.max(sm0, axis=1, keepdims=True)
    sm = sm0
    for _ in range(_NUM_HARD - 1):
        sm = jnp.where(sm < m, sm, -1e9)
        m = jnp.max(sm, axis=1, keepdims=True)

    e = jnp.exp(s)
    e = jnp.where(sm0 >= m, e * e, e)

    rowsum = jnp.sum(e, axis=1)
    part = jnp.sum(jnp.log(rowsum) - 2.0 * s_ii)
    part_vec = jnp.full((1, _LANES), part / _LANES, dtype=jnp.float32)
    colpart = jnp.sum(e, axis=0, keepdims=True)

    @pl.when(i == 0)
    def _init():
        colsum_ref[...] = colpart
        rowpart_ref[...] = part_vec

    @pl.when(i > 0)
    def _acc():
        colsum_ref[...] += colpart
        rowpart_ref[...] += part_vec

    @pl.when(i == n_steps - 1)
    def _final():
        total = (jnp.sum(jnp.log(colsum_ref[...]))
                 + jnp.sum(rowpart_ref[...]))
        out_ref[...] = jnp.full((1, _LANES), total / (2.0 * n_rows),
                                dtype=jnp.float32)


@jax.jit
def kernel(vision_embed, text_embed):
    b, d = vision_embed.shape
    block_r = 256
    grid = (b // block_r,)
    v_lo = (vision_embed * (1.0 / _TEMPERATURE)).astype(jnp.bfloat16)
    t_t = text_embed.T.astype(jnp.bfloat16)

    out = pl.pallas_call(
        functools.partial(_loss_kernel, n_rows=b),
        grid=grid,
        in_specs=[
            pl.BlockSpec((block_r, d), lambda i: (i, 0)),
            pl.BlockSpec((d, b), lambda i: (0, 0)),
        ],
        out_specs=pl.BlockSpec((1, _LANES), lambda i: (0, 0)),
        out_shape=jax.ShapeDtypeStruct((1, _LANES), jnp.float32),
        scratch_shapes=[
            pltpu.VMEM((1, b), jnp.float32),
            pltpu.VMEM((1, _LANES), jnp.float32),
            pltpu.VMEM((block_r, b), jnp.float32),
        ],
    )(v_lo, t_t)
    return out[0, 0]


# speedup vs baseline: 21.2939x; 1.1423x over previous
import functools
import math

import jax
import jax.numpy as jnp
from jax.experimental import pallas as pl
from jax.experimental.pallas import tpu as pltpu

_TEMPERATURE = 0.07
_NUM_HARD = 4
_LANES = 128


def _loss_kernel(v_ref, t_ref, out_ref, colsum_ref, rowpart_ref, e_scr,
                 *, n_rows):
    i = pl.program_id(0)
    n_steps = pl.num_programs(0)

    v = v_ref[...]
    t = t_ref[...]
    s = jax.lax.dot_general(
        v, t, (((1,), (1,)), ((), ())),
        preferred_element_type=jnp.float32)
    r, b = s.shape

    e_scr[...] = jnp.exp2(s)

    col0 = i * r
    w = e_scr[:, pl.ds(col0, r)]
    dmask = (jax.lax.broadcasted_iota(jnp.int32, (r, r), 0)
             == jax.lax.broadcasted_iota(jnp.int32, (r, r), 1))
    wd = jnp.where(dmask, w, 0.0)
    ed_row = jnp.sum(wd, axis=1)
    ed_lane = jnp.sum(wd, axis=0, keepdims=True)
    e_scr[:, pl.ds(col0, r)] = jnp.where(dmask, 0.0, w)
    e0 = e_scr[...]

    m = jnp.max(e0, axis=1, keepdims=True)
    for _ in range(_NUM_HARD - 1):
        m = jnp.max(jnp.where(e0 < m, e0, 0.0), axis=1, keepdims=True)

    esel = jnp.where(e0 >= m, e0 * e0, e0)

    rowsum = jnp.sum(esel, axis=1) + ed_row
    s_ii_sum = jnp.sum(jnp.log(ed_lane))
    part = jnp.sum(jnp.log(rowsum)) - 2.0 * s_ii_sum
    part_vec = jnp.full((1, _LANES), part / _LANES, dtype=jnp.float32)
    ones_row = jnp.ones((1, r), dtype=jnp.float32)
    colpart = jax.lax.dot_general(
        ones_row, esel, (((1,), (0,)), ((), ())),
        preferred_element_type=jnp.float32)

    @pl.when(i == 0)
    def _init():
        colsum_ref[...] = colpart
        rowpart_ref[...] = part_vec

    @pl.when(i > 0)
    def _acc():
        colsum_ref[...] += colpart
        rowpart_ref[...] += part_vec

    colsum_ref[:, pl.ds(col0, r)] += ed_lane

    @pl.when(i == n_steps - 1)
    def _final():
        total = (jnp.sum(jnp.log(colsum_ref[...]))
                 + jnp.sum(rowpart_ref[...]))
        out_ref[...] = jnp.full((1, _LANES), total / (2.0 * n_rows),
                                dtype=jnp.float32)


@jax.jit
def kernel(vision_embed, text_embed):
    b, d = vision_embed.shape
    block_r = 256
    grid = (b // block_r,)
    v_lo = (vision_embed * (math.log2(math.e) / _TEMPERATURE)).astype(
        jnp.bfloat16)
    t_lo = text_embed.astype(jnp.bfloat16)

    out = pl.pallas_call(
        functools.partial(_loss_kernel, n_rows=b),
        grid=grid,
        in_specs=[
            pl.BlockSpec((block_r, d), lambda i: (i, 0)),
            pl.BlockSpec((b, d), lambda i: (0, 0)),
        ],
        out_specs=pl.BlockSpec((1, _LANES), lambda i: (0, 0)),
        out_shape=jax.ShapeDtypeStruct((1, _LANES), jnp.float32),
        scratch_shapes=[
            pltpu.VMEM((1, b), jnp.float32),
            pltpu.VMEM((1, _LANES), jnp.float32),
            pltpu.VMEM((block_r, b), jnp.float32),
        ],
    )(v_lo, t_lo)
    return out[0, 0]


# speedup vs baseline: 23.9937x; 1.1268x over previous
import functools
import math

import jax
import jax.numpy as jnp
from jax.experimental import pallas as pl
from jax.experimental.pallas import tpu as pltpu

_TEMPERATURE = 0.07
_NUM_HARD = 4
_LANES = 128


_SCALE = math.log2(math.e) / _TEMPERATURE


def _loss_kernel(v_ref, t_ref, out_ref, colsum_ref, rowpart_ref, e_scr,
                 t_bf_scr, *, n_rows):
    i = pl.program_id(0)
    n_steps = pl.num_programs(0)

    @pl.when(i == 0)
    def _stage_t():
        t_bf_scr[...] = t_ref[...].astype(jnp.bfloat16)

    v = (v_ref[...] * _SCALE).astype(jnp.bfloat16)
    t = t_bf_scr[...]
    s = jax.lax.dot_general(
        v, t, (((1,), (1,)), ((), ())),
        preferred_element_type=jnp.float32)
    r, b = s.shape

    e_scr[...] = jnp.exp2(s)

    col0 = i * r
    w = e_scr[:, pl.ds(col0, r)]
    dmask = (jax.lax.broadcasted_iota(jnp.int32, (r, r), 0)
             == jax.lax.broadcasted_iota(jnp.int32, (r, r), 1))
    wd = jnp.where(dmask, w, 0.0)
    ed_row = jnp.sum(wd, axis=1)
    ed_lane = jnp.sum(wd, axis=0, keepdims=True)
    e_scr[:, pl.ds(col0, r)] = jnp.where(dmask, 0.0, w)
    e0 = e_scr[...]

    m = jnp.max(e0, axis=1, keepdims=True)
    for _ in range(_NUM_HARD - 1):
        m = jnp.max(jnp.where(e0 < m, e0, 0.0), axis=1, keepdims=True)

    esel = jnp.where(e0 >= m, e0 * e0, e0)

    rowsum = jnp.sum(esel, axis=1) + ed_row
    s_ii_sum = jnp.sum(jnp.log(ed_lane))
    part = jnp.sum(jnp.log(rowsum)) - 2.0 * s_ii_sum
    part_vec = jnp.full((1, _LANES), part / _LANES, dtype=jnp.float32)
    ones_row = jnp.ones((1, r), dtype=jnp.float32)
    colpart = jax.lax.dot_general(
        ones_row, esel, (((1,), (0,)), ((), ())),
        preferred_element_type=jnp.float32)

    @pl.when(i == 0)
    def _init():
        colsum_ref[...] = colpart
        rowpart_ref[...] = part_vec

    @pl.when(i > 0)
    def _acc():
        colsum_ref[...] += colpart
        rowpart_ref[...] += part_vec

    colsum_ref[:, pl.ds(col0, r)] += ed_lane

    @pl.when(i == n_steps - 1)
    def _final():
        total = (jnp.sum(jnp.log(colsum_ref[...]))
                 + jnp.sum(rowpart_ref[...]))
        out_ref[...] = jnp.full((1, _LANES), total / (2.0 * n_rows),
                                dtype=jnp.float32)


@jax.jit
def kernel(vision_embed, text_embed):
    b, d = vision_embed.shape
    block_r = 256
    grid = (b // block_r,)
    out = pl.pallas_call(
        functools.partial(_loss_kernel, n_rows=b),
        grid=grid,
        in_specs=[
            pl.BlockSpec((block_r, d), lambda i: (i, 0)),
            pl.BlockSpec((b, d), lambda i: (0, 0)),
        ],
        out_specs=pl.BlockSpec((1, _LANES), lambda i: (0, 0)),
        out_shape=jax.ShapeDtypeStruct((1, _LANES), jnp.float32),
        scratch_shapes=[
            pltpu.VMEM((1, b), jnp.float32),
            pltpu.VMEM((1, _LANES), jnp.float32),
            pltpu.VMEM((block_r, b), jnp.float32),
            pltpu.VMEM((b, d), jnp.bfloat16),
        ],
    )(vision_embed, text_embed)
    return out[0, 0]
